# baseline (speedup 1.0000x reference)
import functools

import jax
import jax.numpy as jnp
from jax import lax
from jax.experimental import pallas as pl
from jax.experimental.pallas import tpu as pltpu
from jax.experimental.pallas import tpu_sc as plsc

_BATCH = 16384
_D = 32
_NC = 2
_NS = 16
_L = 16
_NW = _NC * _NS
_BW = _BATCH // _NW
_GE = 4
_NST = _BW // _GE
_NROW = 1000000
_BLK = 128
_WD = 0.0001

_mesh = plsc.VectorSubcoreMesh(core_axis_name="c", subcore_axis_name="s")


@functools.partial(
    pl.kernel,
    out_type=(
        jax.ShapeDtypeStruct((_BATCH,), jnp.float32),
        jax.ShapeDtypeStruct((_BATCH,), jnp.float32),
        jax.ShapeDtypeStruct((_NW, _L), jnp.float32),
    ),
    mesh=_mesh,
    compiler_params=pltpu.CompilerParams(
        needs_layout_passes=False, use_tc_tiling_on_sc=True),
    scratch_types=[
        pltpu.VMEM((_BW + _L,), jnp.int32),
        pltpu.VMEM((_BW + _L,), jnp.int32),
        pltpu.VMEM((_BW + _L,), jnp.int32),
        pltpu.VMEM((2, _GE, _D, _BLK), jnp.float32),
        pltpu.VMEM((2, _GE, _D, _BLK), jnp.float32),
        pltpu.VMEM((2, _GE, _D, _BLK), jnp.float32),
        pltpu.VMEM((_BW,), jnp.float32),
        pltpu.VMEM((_BW,), jnp.float32),
        pltpu.VMEM((_L,), jnp.float32),
        pltpu.SemaphoreType.DMA((2, _GE)),
        pltpu.SemaphoreType.DMA,
    ],
)
def _sc_gather_dot(users, pos, neg, ue_t, ie_t,
                   s_pos_out, s_neg_out, sq_out,
                   idx_u, idx_p, idx_n, bu, bp, bn,
                   sp_v, sn_v, sq_v, sems, sem):
    wid = lax.axis_index("s") * _NC + lax.axis_index("c")
    base = wid * _BW

    pltpu.sync_copy(users.at[pl.ds(base, _BW)], idx_u.at[pl.ds(0, _BW)])
    pltpu.sync_copy(pos.at[pl.ds(base, _BW)], idx_p.at[pl.ds(0, _BW)])
    pltpu.sync_copy(neg.at[pl.ds(base, _BW)], idx_n.at[pl.ds(0, _BW)])

    d_lo = lax.iota(jnp.int32, _L)
    d_hi = d_lo + _L
    max_off = _NROW - _BLK
    dummy = ue_t.at[:, pl.ds(0, _BLK)]

    def block_off(r):
        boff = jnp.minimum(r - lax.rem(r, _BLK), max_off)
        return pl.multiple_of(boff, _BLK)

    def fire(step, buf):
        e0 = step * _GE
        vu = idx_u[pl.ds(e0, _L)]
        vp = idx_p[pl.ds(e0, _L)]
        vn = idx_n[pl.ds(e0, _L)]
        for t in range(_GE):
            for vec, tbl, dst in ((vu, ue_t, bu), (vp, ie_t, bp), (vn, ie_t, bn)):
                boff = block_off(vec[t])
                for k in range(_D // 8):
                    pltpu.async_copy(
                        tbl.at[pl.ds(k * 8, 8), pl.ds(boff, _BLK)],
                        dst.at[buf, t, pl.ds(k * 8, 8)], sems.at[buf, t])

    def consume(step, buf, sq_acc):
        e0 = step * _GE
        vu = idx_u[pl.ds(e0, _L)]
        vp = idx_p[pl.ds(e0, _L)]
        vn = idx_n[pl.ds(e0, _L)]
        s_pos = []
        s_neg = []
        for t in range(_GE):
            for dst in (bu, bp, bn):
                pltpu.make_async_copy(dummy, dst.at[buf, t],
                                      sems.at[buf, t]).wait()
            lu = vu[t] - block_off(vu[t])
            lp = vp[t] - block_off(vp[t])
            ln = vn[t] - block_off(vn[t])
            u0 = plsc.load_gather(bu.at[buf, t], [d_lo, jnp.full((_L,), 0, jnp.int32) + lu])
            u1 = plsc.load_gather(bu.at[buf, t], [d_hi, jnp.full((_L,), 0, jnp.int32) + lu])
            p0 = plsc.load_gather(bp.at[buf, t], [d_lo, jnp.full((_L,), 0, jnp.int32) + lp])
            p1 = plsc.load_gather(bp.at[buf, t], [d_hi, jnp.full((_L,), 0, jnp.int32) + lp])
            n0 = plsc.load_gather(bn.at[buf, t], [d_lo, jnp.full((_L,), 0, jnp.int32) + ln])
            n1 = plsc.load_gather(bn.at[buf, t], [d_hi, jnp.full((_L,), 0, jnp.int32) + ln])
            s_pos.append(jnp.sum(u0 * p0 + u1 * p1))
            s_neg.append(jnp.sum(u0 * n0 + u1 * n1))
            sq_acc = sq_acc + (u0 * u0 + u1 * u1 + p0 * p0 + p1 * p1
                               + n0 * n0 + n1 * n1)
        pos_vec = s_pos[-1]
        neg_vec = s_neg[-1]
        for t in range(_GE - 2, -1, -1):
            pos_vec = jnp.where(d_lo == t, s_pos[t], pos_vec)
            neg_vec = jnp.where(d_lo == t, s_neg[t], neg_vec)
        mask = d_lo < _GE
        plsc.store_scatter(sp_v, [e0 + d_lo], pos_vec, mask=mask)
        plsc.store_scatter(sn_v, [e0 + d_lo], neg_vec, mask=mask)
        return sq_acc

    fire(0, 0)

    def body(i, sq_acc):
        fire(2 * i + 1, 1)
        sq_acc = consume(2 * i, 0, sq_acc)

        @pl.when(i < _NST // 2 - 1)
        def _():
            fire(2 * i + 2, 0)

        return consume(2 * i + 1, 1, sq_acc)

    sq_acc = lax.fori_loop(0, _NST // 2, body, jnp.zeros((_L,), jnp.float32))
    sq_v[...] = sq_acc

    pltpu.sync_copy(sp_v, s_pos_out.at[pl.ds(base, _BW)])
    pltpu.sync_copy(sn_v, s_neg_out.at[pl.ds(base, _BW)])
    pltpu.sync_copy(sq_v, sq_out.at[wid])


def _tc_loss_body(pos_ref, neg_ref, sq_ref, out_ref):
    sp = pos_ref[...]
    sn = neg_ref[...]
    bce = jnp.sum(jnp.log(1.0 + jnp.exp(-sp))) + jnp.sum(jnp.log(1.0 + jnp.exp(sn)))
    reg = jnp.sum(sq_ref[...])
    out_ref[0, 0] = bce / (2.0 * _BATCH) + _WD * 0.5 * reg / _BATCH


_tc_loss = pl.pallas_call(
    _tc_loss_body,
    out_shape=jax.ShapeDtypeStruct((1, 1), jnp.float32),
    out_specs=pl.BlockSpec(memory_space=pltpu.SMEM),
)


def kernel(users, positive_items, negative_items, user_embedding, item_embedding):
    s_pos, s_neg, sq = _sc_gather_dot(
        users, positive_items, negative_items,
        user_embedding.T, item_embedding.T)
    out = _tc_loss(s_pos.reshape(128, 128), s_neg.reshape(128, 128),
                   sq.reshape(4, 128))
    return out.reshape(())


# speedup vs baseline: 1.0013x; 1.0013x over previous
import functools

import jax
import jax.numpy as jnp
from jax import lax
from jax.experimental import pallas as pl
from jax.experimental.pallas import tpu as pltpu
from jax.experimental.pallas import tpu_sc as plsc

_BATCH = 16384
_D = 32
_NC = 2
_NS = 16
_L = 16
_NW = _NC * _NS
_BW = _BATCH // _NW
_GE = 4
_NST = _BW // _GE
_NROW = 1000000
_BLK = 128
_WD = 0.0001

_mesh = plsc.VectorSubcoreMesh(core_axis_name="c", subcore_axis_name="s")


@functools.partial(
    pl.kernel,
    out_type=(
        jax.ShapeDtypeStruct((_BATCH,), jnp.float32),
        jax.ShapeDtypeStruct((_BATCH,), jnp.float32),
        jax.ShapeDtypeStruct((_NW, _L), jnp.float32),
    ),
    mesh=_mesh,
    compiler_params=pltpu.CompilerParams(
        needs_layout_passes=False, use_tc_tiling_on_sc=True),
    scratch_types=[
        pltpu.VMEM((_BW + _L,), jnp.int32),
        pltpu.VMEM((_BW + _L,), jnp.int32),
        pltpu.VMEM((_BW + _L,), jnp.int32),
        pltpu.VMEM((2, _GE, _D, _BLK), jnp.float32),
        pltpu.VMEM((2, _GE, _D, _BLK), jnp.float32),
        pltpu.VMEM((2, _GE, _D, _BLK), jnp.float32),
        pltpu.VMEM((_BW,), jnp.float32),
        pltpu.VMEM((_BW,), jnp.float32),
        pltpu.VMEM((_L,), jnp.float32),
        pltpu.SemaphoreType.DMA((2, _GE)),
        pltpu.SemaphoreType.DMA,
    ],
)
def _sc_gather_dot(users, pos, neg, ue_t, ie_t,
                   s_pos_out, s_neg_out, sq_out,
                   idx_u, idx_p, idx_n, bu, bp, bn,
                   sp_v, sn_v, sq_v, sems, sem):
    wid = lax.axis_index("s") * _NC + lax.axis_index("c")
    base = wid * _BW

    pltpu.sync_copy(users.at[pl.ds(base, _BW)], idx_u.at[pl.ds(0, _BW)])
    pltpu.sync_copy(pos.at[pl.ds(base, _BW)], idx_p.at[pl.ds(0, _BW)])
    pltpu.sync_copy(neg.at[pl.ds(base, _BW)], idx_n.at[pl.ds(0, _BW)])

    d_lo = lax.iota(jnp.int32, _L)
    d_hi = d_lo + _L
    max_off = _NROW - _BLK
    dummy = ue_t.at[:, pl.ds(0, _BLK)]

    def block_off(r):
        boff = jnp.minimum(r - lax.rem(r, _BLK), max_off)
        return pl.multiple_of(boff, _BLK)

    def fire(step, buf):
        e0 = step * _GE
        vu = idx_u[pl.ds(e0, _L)]
        vp = idx_p[pl.ds(e0, _L)]
        vn = idx_n[pl.ds(e0, _L)]
        for t in range(_GE):
            for vec, tbl, dst in ((vu, ue_t, bu), (vp, ie_t, bp), (vn, ie_t, bn)):
                pltpu.async_copy(
                    tbl.at[:, pl.ds(block_off(vec[t]), _BLK)],
                    dst.at[buf, t], sems.at[buf, t])

    def consume(step, buf, sq_acc):
        e0 = step * _GE
        vu = idx_u[pl.ds(e0, _L)]
        vp = idx_p[pl.ds(e0, _L)]
        vn = idx_n[pl.ds(e0, _L)]
        s_pos = []
        s_neg = []
        for t in range(_GE):
            for dst in (bu, bp, bn):
                pltpu.make_async_copy(dummy, dst.at[buf, t],
                                      sems.at[buf, t]).wait()
            lu = vu[t] - block_off(vu[t])
            lp = vp[t] - block_off(vp[t])
            ln = vn[t] - block_off(vn[t])
            u0 = plsc.load_gather(bu.at[buf, t], [d_lo, jnp.full((_L,), 0, jnp.int32) + lu])
            u1 = plsc.load_gather(bu.at[buf, t], [d_hi, jnp.full((_L,), 0, jnp.int32) + lu])
            p0 = plsc.load_gather(bp.at[buf, t], [d_lo, jnp.full((_L,), 0, jnp.int32) + lp])
            p1 = plsc.load_gather(bp.at[buf, t], [d_hi, jnp.full((_L,), 0, jnp.int32) + lp])
            n0 = plsc.load_gather(bn.at[buf, t], [d_lo, jnp.full((_L,), 0, jnp.int32) + ln])
            n1 = plsc.load_gather(bn.at[buf, t], [d_hi, jnp.full((_L,), 0, jnp.int32) + ln])
            s_pos.append(jnp.sum(u0 * p0 + u1 * p1))
            s_neg.append(jnp.sum(u0 * n0 + u1 * n1))
            sq_acc = sq_acc + (u0 * u0 + u1 * u1 + p0 * p0 + p1 * p1
                               + n0 * n0 + n1 * n1)
        pos_vec = s_pos[-1]
        neg_vec = s_neg[-1]
        for t in range(_GE - 2, -1, -1):
            pos_vec = jnp.where(d_lo == t, s_pos[t], pos_vec)
            neg_vec = jnp.where(d_lo == t, s_neg[t], neg_vec)
        mask = d_lo < _GE
        plsc.store_scatter(sp_v, [e0 + d_lo], pos_vec, mask=mask)
        plsc.store_scatter(sn_v, [e0 + d_lo], neg_vec, mask=mask)
        return sq_acc

    fire(0, 0)

    def body(i, sq_acc):
        fire(2 * i + 1, 1)
        sq_acc = consume(2 * i, 0, sq_acc)

        @pl.when(i < _NST // 2 - 1)
        def _():
            fire(2 * i + 2, 0)

        return consume(2 * i + 1, 1, sq_acc)

    sq_acc = lax.fori_loop(0, _NST // 2, body, jnp.zeros((_L,), jnp.float32))
    sq_v[...] = sq_acc

    pltpu.sync_copy(sp_v, s_pos_out.at[pl.ds(base, _BW)])
    pltpu.sync_copy(sn_v, s_neg_out.at[pl.ds(base, _BW)])
    pltpu.sync_copy(sq_v, sq_out.at[wid])


def _tc_loss_body(pos_ref, neg_ref, sq_ref, out_ref):
    sp = pos_ref[...]
    sn = neg_ref[...]
    bce = jnp.sum(jnp.log(1.0 + jnp.exp(-sp))) + jnp.sum(jnp.log(1.0 + jnp.exp(sn)))
    reg = jnp.sum(sq_ref[...])
    out_ref[0, 0] = bce / (2.0 * _BATCH) + _WD * 0.5 * reg / _BATCH


_tc_loss = pl.pallas_call(
    _tc_loss_body,
    out_shape=jax.ShapeDtypeStruct((1, 1), jnp.float32),
    out_specs=pl.BlockSpec(memory_space=pltpu.SMEM),
)


def kernel(users, positive_items, negative_items, user_embedding, item_embedding):
    s_pos, s_neg, sq = _sc_gather_dot(
        users, positive_items, negative_items,
        user_embedding.T, item_embedding.T)
    out = _tc_loss(s_pos.reshape(128, 128), s_neg.reshape(128, 128),
                   sq.reshape(4, 128))
    return out.reshape(())
